# baseline (speedup 1.0000x reference)
import functools

import jax
import jax.numpy as jnp
from jax import lax
from jax.experimental import pallas as pl
from jax.experimental.pallas import tpu as pltpu
from jax.experimental.pallas import tpu_sc as plsc

_B, _C, _N = 8, 32, 2048
_OUT, _SR, _K = 64, 2, 4
_TN = 1024
_NT = _N // _TN
_F32 = jnp.float32
_I32 = jnp.int32


def _tc1_body(x_full_ref, x_tile_ref, idx_ref):
  xb = x_full_ref[0]
  xt = x_tile_ref[0]

  col_n2 = jnp.sum(xb * xb, axis=0, keepdims=True)
  row_n2 = jnp.sum(xt * xt, axis=0)[:, None]
  prod = jax.lax.dot_general(xt, xb, (((0,), (0,)), ((), ())),
                             preferred_element_type=_F32)
  inner = -2.0 * prod
  dist = -col_n2 - inner - row_n2

  iota = jax.lax.broadcasted_iota(_I32, (_TN, _N), 1)
  base = pl.program_id(0) * _N

  cols = []
  for _ in range(_K):
    idx = jnp.argmax(dist, axis=1)[:, None]
    dist = jnp.where(iota == idx, -jnp.inf, dist)
    cols.append(idx + base)
  idxcat = jnp.concatenate(cols + cols, axis=1)
  idx_ref[0] = jnp.transpose(idxcat, (1, 0))


def _tc2_body(xT_ref, g0_ref, g1_ref, g2_ref, g3_ref,
              W1_ref, b1_ref, W2_ref, b2_ref, W3blk_ref, b3blk_ref, out_ref):
  xtr = xT_ref[0]
  acc = None
  for g_ref in (g0_ref, g1_ref, g2_ref, g3_ref):
    xg = g_ref[0, 0][:, :_C]
    feat = jnp.concatenate([xtr, xg], axis=1)
    e1 = jax.lax.dot_general(feat, W1_ref[...], (((1,), (1,)), ((), ())),
                             preferred_element_type=_F32)
    e1 = e1 + b1_ref[...]
    g = jax.nn.relu(jnp.concatenate([e1, feat], axis=1))
    e2 = jax.lax.dot_general(g, W2_ref[...], (((1,), (1,)), ((), ())),
                             preferred_element_type=_F32)
    e2 = jax.nn.relu(e2 + b2_ref[...])
    h = jax.lax.dot_general(e2, W3blk_ref[...], (((1,), (0,)), ((), ())),
                            preferred_element_type=_F32)
    h = h + b3blk_ref[...]
    acc = h if acc is None else jnp.maximum(acc, h)
  out_ref[0] = acc


_D = 128
_CH = 512


def _make_sc_gather():
  info = plsc.get_sparse_core_info()
  nc, ns = info.num_cores, info.num_subcores
  nw = nc * ns
  tot = _B * _K * _N
  per_w = tot // nw
  mesh = plsc.VectorSubcoreMesh(core_axis_name="c", subcore_axis_name="s")

  @functools.partial(
      pl.kernel, mesh=mesh,
      out_type=jax.ShapeDtypeStruct((tot, _D), _F32),
      scratch_types=[
          pltpu.VMEM((per_w,), _I32),
          pltpu.VMEM((_CH, _D), _F32),
          pltpu.SemaphoreType.DMA,
      ],
  )
  def sc_gather(table_hbm, idx_hbm, out_hbm, idx_v, rows_v, sem):
    wid = lax.axis_index("s") * nc + lax.axis_index("c")
    base = wid * per_w
    pltpu.sync_copy(idx_hbm.at[pl.ds(base, per_w)], idx_v)
    for c in range(per_w // _CH):
      pltpu.async_copy(
          table_hbm.at[idx_v.at[pl.ds(c * _CH, _CH)]], rows_v, sem).wait()
      pltpu.sync_copy(rows_v, out_hbm.at[pl.ds(base + c * _CH, _CH)])

  return sc_gather


@jax.jit
def kernel(x, W1, b1, W2, b2, W3, b3):
  b1r = b1.reshape(1, _OUT)
  b2r = b2.reshape(1, _OUT * _SR)
  z = jnp.zeros((_OUT, _OUT), _F32)
  W3blk = jnp.block([[W3.T, z], [z, W3.T]])
  b3blk = jnp.concatenate([b3, b3]).reshape(1, _OUT * _SR)

  idx8 = pl.pallas_call(
      _tc1_body,
      grid=(_B, _NT),
      in_specs=[
          pl.BlockSpec((1, _C, _N), lambda b, t: (b, 0, 0)),
          pl.BlockSpec((1, _C, _TN), lambda b, t: (b, 0, t)),
      ],
      out_specs=pl.BlockSpec((1, 8, _TN), lambda b, t: (b, 0, t)),
      out_shape=jax.ShapeDtypeStruct((_B, 8, _N), _I32),
      compiler_params=pltpu.CompilerParams(
          dimension_semantics=("parallel", "parallel")),
  )(x, x)

  idx_flat = idx8[:, :_K, :].reshape(_B * _K * _N)
  table = jnp.zeros((_B * _N, _D), _F32).at[:, :_C].set(
      jnp.transpose(x, (0, 2, 1)).reshape(_B * _N, _C))
  gathered = _make_sc_gather()(table, idx_flat)
  g4 = gathered.reshape(_B, _K, _N, _D)

  out = pl.pallas_call(
      _tc2_body,
      grid=(_B, _NT),
      in_specs=[
          pl.BlockSpec((1, _TN, _C), lambda b, t: (b, t, 0)),
          pl.BlockSpec((1, 1, _TN, _D), lambda b, t: (b, 0, t, 0)),
          pl.BlockSpec((1, 1, _TN, _D), lambda b, t: (b, 1, t, 0)),
          pl.BlockSpec((1, 1, _TN, _D), lambda b, t: (b, 2, t, 0)),
          pl.BlockSpec((1, 1, _TN, _D), lambda b, t: (b, 3, t, 0)),
          pl.BlockSpec((_OUT, 2 * _C), lambda b, t: (0, 0)),
          pl.BlockSpec((1, _OUT), lambda b, t: (0, 0)),
          pl.BlockSpec((_OUT * _SR, 2 * _C + _OUT), lambda b, t: (0, 0)),
          pl.BlockSpec((1, _OUT * _SR), lambda b, t: (0, 0)),
          pl.BlockSpec((_OUT * _SR, _OUT * _SR), lambda b, t: (0, 0)),
          pl.BlockSpec((1, _OUT * _SR), lambda b, t: (0, 0)),
      ],
      out_specs=pl.BlockSpec((1, _TN, _OUT * _SR), lambda b, t: (b, t, 0)),
      out_shape=jax.ShapeDtypeStruct((_B, _N, _OUT * _SR), _F32),
      compiler_params=pltpu.CompilerParams(
          dimension_semantics=("parallel", "parallel")),
  )(jnp.transpose(x, (0, 2, 1)), g4, g4, g4, g4,
    W1, b1r, W2, b2r, W3blk, b3blk)

  return (out.reshape(_B, _N, _SR, _OUT)
          .transpose(0, 3, 1, 2)
          .reshape(_B, _OUT, _N * _SR))


# speedup vs baseline: 1.0851x; 1.0851x over previous
import functools

import jax
import jax.numpy as jnp
from jax import lax
from jax.experimental import pallas as pl
from jax.experimental.pallas import tpu as pltpu
from jax.experimental.pallas import tpu_sc as plsc

_B, _C, _N = 8, 32, 2048
_OUT, _SR, _K = 64, 2, 4
_TN = 1024
_NT = _N // _TN
_HB = _B // 2
_F32 = jnp.float32
_I32 = jnp.int32

_D = 128
_CH = 512


def _tc1_body(x_full_ref, x_tile_ref, idx_ref, tab_ref):
  xb = x_full_ref[0]
  xt = x_tile_ref[0]

  col_n2 = jnp.sum(xb * xb, axis=0, keepdims=True)
  row_n2 = jnp.sum(xt * xt, axis=0)[:, None]
  prod = jax.lax.dot_general(xt, xb, (((0,), (0,)), ((), ())),
                             preferred_element_type=_F32)
  inner = -2.0 * prod
  dist = -col_n2 - inner - row_n2

  iota = jax.lax.broadcasted_iota(_I32, (_TN, _N), 1)
  base = pl.program_id(0) * _N

  cols = []
  for _ in range(_K):
    idx = jnp.argmax(dist, axis=1)[:, None]
    dist = jnp.where(iota == idx, -jnp.inf, dist)
    cols.append(idx + base)
  idxcat = jnp.concatenate(cols + cols, axis=1)
  idx_ref[0] = jnp.transpose(idxcat, (1, 0))
  tab_ref[0] = jnp.concatenate(
      [jnp.transpose(xt, (1, 0)), jnp.zeros((_TN, _D - _C), _F32)], axis=1)


def _tc2_body(xT_ref, g0_ref, g1_ref, g2_ref, g3_ref,
              W1_ref, b1_ref, W2_ref, b2_ref, W3blk_ref, b3blk_ref, out_ref):
  xtr = xT_ref[0][:, :_C]
  acc = None
  for g_ref in (g0_ref, g1_ref, g2_ref, g3_ref):
    xg = g_ref[0, 0][:, :_C]
    feat = jnp.concatenate([xtr, xg], axis=1)
    e1 = jax.lax.dot_general(feat, W1_ref[...], (((1,), (1,)), ((), ())),
                             preferred_element_type=_F32)
    e1 = e1 + b1_ref[...]
    g = jax.nn.relu(jnp.concatenate([e1, feat], axis=1))
    e2 = jax.lax.dot_general(g, W2_ref[...], (((1,), (1,)), ((), ())),
                             preferred_element_type=_F32)
    e2 = jax.nn.relu(e2 + b2_ref[...])
    h = jax.lax.dot_general(e2, W3blk_ref[...], (((1,), (0,)), ((), ())),
                            preferred_element_type=_F32)
    h = h + b3blk_ref[...]
    acc = h if acc is None else jnp.maximum(acc, h)
  out_ref[0] = acc


def _make_sc_gather(tot):
  info = plsc.get_sparse_core_info()
  nc, ns = info.num_cores, info.num_subcores
  nw = nc * ns
  per_w = tot // nw
  mesh = plsc.VectorSubcoreMesh(core_axis_name="c", subcore_axis_name="s")

  @functools.partial(
      pl.kernel, mesh=mesh,
      out_type=jax.ShapeDtypeStruct((tot, _D), _F32),
      scratch_types=[
          pltpu.VMEM((per_w,), _I32),
          pltpu.VMEM((_CH, _D), _F32),
          pltpu.SemaphoreType.DMA,
      ],
  )
  def sc_gather(table_hbm, idx_hbm, out_hbm, idx_v, rows_v, sem):
    wid = lax.axis_index("s") * nc + lax.axis_index("c")
    base = wid * per_w
    pltpu.sync_copy(idx_hbm.at[pl.ds(base, per_w)], idx_v)
    for c in range(per_w // _CH):
      pltpu.async_copy(
          table_hbm.at[idx_v.at[pl.ds(c * _CH, _CH)]], rows_v, sem).wait()
      pltpu.sync_copy(rows_v, out_hbm.at[pl.ds(base + c * _CH, _CH)])

  return sc_gather


def _half(xh, W1, b1r, W2, b2r, W3blk, b3blk):
  idx8, tab = pl.pallas_call(
      _tc1_body,
      grid=(_HB, _NT),
      in_specs=[
          pl.BlockSpec((1, _C, _N), lambda b, t: (b, 0, 0)),
          pl.BlockSpec((1, _C, _TN), lambda b, t: (b, 0, t)),
      ],
      out_specs=[
          pl.BlockSpec((1, 8, _TN), lambda b, t: (b, 0, t)),
          pl.BlockSpec((1, _TN, _D), lambda b, t: (b, t, 0)),
      ],
      out_shape=[
          jax.ShapeDtypeStruct((_HB, 8, _N), _I32),
          jax.ShapeDtypeStruct((_HB, _N, _D), _F32),
      ],
      compiler_params=pltpu.CompilerParams(
          dimension_semantics=("parallel", "parallel")),
  )(xh, xh)

  idx_flat = idx8[:, :_K, :].reshape(_HB * _K * _N)
  gathered = _make_sc_gather(_HB * _K * _N)(
      tab.reshape(_HB * _N, _D), idx_flat)
  g4 = gathered.reshape(_HB, _K, _N, _D)

  return pl.pallas_call(
      _tc2_body,
      grid=(_HB, _NT),
      in_specs=[
          pl.BlockSpec((1, _TN, _D), lambda b, t: (b, t, 0)),

          pl.BlockSpec((1, 1, _TN, _D), lambda b, t: (b, 0, t, 0)),
          pl.BlockSpec((1, 1, _TN, _D), lambda b, t: (b, 1, t, 0)),
          pl.BlockSpec((1, 1, _TN, _D), lambda b, t: (b, 2, t, 0)),
          pl.BlockSpec((1, 1, _TN, _D), lambda b, t: (b, 3, t, 0)),
          pl.BlockSpec((_OUT, 2 * _C), lambda b, t: (0, 0)),
          pl.BlockSpec((1, _OUT), lambda b, t: (0, 0)),
          pl.BlockSpec((_OUT * _SR, 2 * _C + _OUT), lambda b, t: (0, 0)),
          pl.BlockSpec((1, _OUT * _SR), lambda b, t: (0, 0)),
          pl.BlockSpec((_OUT * _SR, _OUT * _SR), lambda b, t: (0, 0)),
          pl.BlockSpec((1, _OUT * _SR), lambda b, t: (0, 0)),
      ],
      out_specs=pl.BlockSpec((1, _TN, _OUT * _SR), lambda b, t: (b, t, 0)),
      out_shape=jax.ShapeDtypeStruct((_HB, _N, _OUT * _SR), _F32),
      compiler_params=pltpu.CompilerParams(
          dimension_semantics=("parallel", "parallel")),
  )(tab, g4, g4, g4, g4,
    W1, b1r, W2, b2r, W3blk, b3blk)


@jax.jit
def kernel(x, W1, b1, W2, b2, W3, b3):
  b1r = b1.reshape(1, _OUT)
  b2r = b2.reshape(1, _OUT * _SR)
  z = jnp.zeros((_OUT, _OUT), _F32)
  W3blk = jnp.block([[W3.T, z], [z, W3.T]])
  b3blk = jnp.concatenate([b3, b3]).reshape(1, _OUT * _SR)

  outs = [_half(x[h * _HB:(h + 1) * _HB], W1, b1r, W2, b2r, W3blk, b3blk)
          for h in range(2)]
  out = jnp.concatenate(outs, axis=0)

  return (out.reshape(_B, _N, _SR, _OUT)
          .transpose(0, 3, 1, 2)
          .reshape(_B, _OUT, _N * _SR))
